# baseline (speedup 1.0000x reference)
import functools

import jax
import jax.numpy as jnp
from jax import lax
from jax.experimental import pallas as pl
from jax.experimental.pallas import tpu as pltpu
from jax.experimental.pallas import tpu_sc as plsc

N = 10000
NP = 10240
E = 320000
IN_CH = 128
HID_CH = 128
OUT_CH = 64

NC = 2
NS = 16
NW = NC * NS
L = 16

BLK = 128
EP = 327680
EBT = EP // (NW * BLK)
ROWS_PT = NP // NS

_mesh = plsc.VectorSubcoreMesh(core_axis_name="c", subcore_axis_name="s")
_SC_PARAMS = pltpu.CompilerParams(use_tc_tiling_on_sc=False,
                                  needs_layout_passes=False)


NPH = NP // 2
HWID = NPH + L


@functools.partial(
    pl.kernel,
    out_type=jax.ShapeDtypeStruct((NW, NP), jnp.float32),
    mesh=_mesh,
    compiler_params=_SC_PARAMS,
    scratch_types=[
        pltpu.VMEM((EBT, BLK), jnp.int32),
        pltpu.VMEM((L, HWID), jnp.float32),
        pltpu.VMEM((NP,), jnp.float32),
    ],
)
def _deg_kernel(dst_hbm, deg_out, idx_v, hist_v, out_v):
    c = lax.axis_index("c")
    s = lax.axis_index("s")
    wid = s * NC + c
    pltpu.sync_copy(dst_hbm.at[wid], idx_v)
    lanes = lax.iota(jnp.int32, L)
    ones = jnp.ones((L,), jnp.float32)
    zeros = jnp.zeros((L,), jnp.float32)
    for p in range(2):
        base = p * NPH

        @pl.loop(0, L)
        def _(r):
            @pl.loop(0, HWID, step=L)
            def _(i):
                hist_v[r, pl.ds(i, L)] = zeros

        @pl.loop(0, EBT)
        def _(j):
            @pl.loop(0, BLK, step=L)
            def _(g):
                v = idx_v[j, pl.ds(g, L)] - base
                inr = (v >= 0) & (v < NPH)
                w = jnp.where(inr, v, NPH)
                plsc.addupdate_scatter(hist_v, [lanes, w], ones)

        @pl.loop(0, NPH, step=L)
        def _(i):
            acc = hist_v[0, pl.ds(i, L)]
            for r in range(1, L):
                acc = acc + hist_v[r, pl.ds(i, L)]
            out_v[pl.ds(base + i, L)] = acc

    pltpu.sync_copy(out_v, deg_out.at[wid])


def _make_prop(D, NBUF):
    @functools.partial(
        pl.kernel,
        out_type=jax.ShapeDtypeStruct((NC, NP, D), jnp.float32),
        mesh=_mesh,
        compiler_params=_SC_PARAMS,
        scratch_types=(
            [pltpu.VMEM((EBT, BLK), jnp.int32),
             pltpu.VMEM((EBT, BLK), jnp.int32)]
            + [pltpu.VMEM((BLK, D), jnp.float32)] * NBUF
            + [pltpu.VMEM_SHARED((NP, D), jnp.float32)]
            + [pltpu.SemaphoreType.DMA] * (2 * NBUF)
        ),
    )
    def _prop(u_hbm, src_hbm, dst_hbm, zeros_hbm, acc_out, si_v, di_v, *bufs):
        rows = bufs[:NBUF]
        acc_sh = bufs[NBUF]
        sem_g = bufs[NBUF + 1:2 * NBUF + 1]
        sem_s = bufs[2 * NBUF + 1:3 * NBUF + 1]
        c = lax.axis_index("c")
        s = lax.axis_index("s")
        wid = s * NC + c
        row0 = s * ROWS_PT
        pltpu.sync_copy(zeros_hbm.at[pl.ds(row0, ROWS_PT)],
                        acc_sh.at[pl.ds(row0, ROWS_PT)])
        pltpu.sync_copy(src_hbm.at[wid], si_v)
        pltpu.sync_copy(dst_hbm.at[wid], di_v)
        plsc.subcore_barrier()

        for b in range(NBUF):
            pltpu.async_copy(u_hbm.at[si_v.at[b]], rows[b], sem_g[b])

        @pl.loop(0, EBT - NBUF, step=NBUF)
        def _(j0):
            scatters = []
            for b in range(NBUF):
                jb = j0 + b
                pltpu.make_async_copy(u_hbm.at[si_v.at[jb]],
                                      rows[b], sem_g[b]).wait()
                scatters.append(
                    pltpu.async_copy(rows[b], acc_sh.at[di_v.at[jb]],
                                     sem_s[b], add=True))
            for b in range(NBUF):
                scatters[b].wait()
                pltpu.async_copy(u_hbm.at[si_v.at[j0 + b + NBUF]],
                                 rows[b], sem_g[b])

        for b in range(NBUF):
            jb = EBT - NBUF + b
            pltpu.make_async_copy(u_hbm.at[si_v.at[jb]],
                                  rows[b], sem_g[b]).wait()
            pltpu.async_copy(rows[b], acc_sh.at[di_v.at[jb]],
                             sem_s[b], add=True).wait()

        plsc.subcore_barrier()
        pltpu.sync_copy(acc_sh.at[pl.ds(row0, ROWS_PT)],
                        acc_out.at[c].at[pl.ds(row0, ROWS_PT)])

    return _prop


_prop64 = _make_prop(OUT_CH, 8)


@functools.partial(
    pl.kernel,
    out_type=jax.ShapeDtypeStruct((NW, EBT, BLK), jnp.float32),
    mesh=_mesh,
    compiler_params=_SC_PARAMS,
    scratch_types=(
        [pltpu.VMEM((EBT, BLK), jnp.int32),
         pltpu.VMEM((EBT, BLK), jnp.int32),
         pltpu.VMEM((EBT, BLK), jnp.float32)]
        + [pltpu.VMEM((BLK, OUT_CH), jnp.float32)] * 8
        + [pltpu.SemaphoreType.DMA] * 8
    ),
)
def _decode_kernel(z2_hbm, s_hbm, d_hbm, out_hbm, si_v, di_v, o_v, *bufs):
    DB = 4
    zs = bufs[:DB]
    zd = bufs[DB:2 * DB]
    sem_s = bufs[2 * DB:3 * DB]
    sem_d = bufs[3 * DB:]
    c = lax.axis_index("c")
    s = lax.axis_index("s")
    wid = s * NC + c
    pltpu.sync_copy(s_hbm.at[wid], si_v)
    pltpu.sync_copy(d_hbm.at[wid], di_v)
    row0 = s * ROWS_PT

    def scoped(z2_sh):
        last_lane = lax.iota(jnp.int32, L) == (L - 1)

        def compute_block(jb, b):
            jb_vec = jnp.full((L,), jb, jnp.int32)

            @pl.loop(0, BLK, step=8)
            def _(e0):
                e0_vec = jnp.full((L,), e0, jnp.int32)
                for k in range(8):
                    e = e0 + k
                    p = zs[b][e, pl.ds(0, L)] * zd[b][e, pl.ds(0, L)]
                    for q in range(1, OUT_CH // L):
                        p = p + (zs[b][e, pl.ds(q * L, L)]
                                 * zd[b][e, pl.ds(q * L, L)])
                    tot = plsc.cumsum(p)
                    plsc.store_scatter(o_v, [jb_vec, e0_vec + k],
                                       tot, mask=last_lane)

        def wait_block(jb, b):
            pltpu.make_async_copy(z2_sh.at[si_v.at[jb]], zs[b],
                                  sem_s[b]).wait()
            pltpu.make_async_copy(z2_sh.at[di_v.at[jb]], zd[b],
                                  sem_d[b]).wait()

        for b in range(DB):
            pltpu.async_copy(z2_sh.at[si_v.at[b]], zs[b], sem_s[b])
            pltpu.async_copy(z2_sh.at[di_v.at[b]], zd[b], sem_d[b])

        @pl.loop(0, EBT - DB, step=DB)
        def _(j0):
            for b in range(DB):
                jb = j0 + b
                wait_block(jb, b)
                compute_block(jb, b)
                jn = jb + DB
                pltpu.async_copy(z2_sh.at[si_v.at[jn]], zs[b], sem_s[b])
                pltpu.async_copy(z2_sh.at[di_v.at[jn]], zd[b], sem_d[b])

        for b in range(DB):
            jb = EBT - DB + b
            wait_block(jb, b)
            compute_block(jb, b)

        pltpu.sync_copy(o_v, out_hbm.at[wid])

    scoped(z2_hbm)


RB = 512
_GRID = (NP // RB,)
_HI = jax.lax.Precision.HIGHEST


def _mm1_body(deg_ref, x_ref, w1_ref, dinv_ref, u1a_ref, u1b_ref):
    deg = jnp.sum(deg_ref[...], axis=0) + 1.0
    dinv = jnp.broadcast_to(jax.lax.rsqrt(deg)[:, None], (RB, L))
    dinv_ref[...] = dinv
    xw = jnp.dot(x_ref[...], w1_ref[...],
                 preferred_element_type=jnp.float32)
    u1 = dinv[:, :1] * xw
    u1a_ref[...] = u1[:, :OUT_CH]
    u1b_ref[...] = u1[:, OUT_CH:]


_mm1 = pl.pallas_call(
    _mm1_body,
    grid=_GRID,
    in_specs=[
        pl.BlockSpec((NW, RB), lambda i: (0, i)),
        pl.BlockSpec((RB, IN_CH), lambda i: (i, 0)),
        pl.BlockSpec((IN_CH, HID_CH), lambda i: (0, 0)),
    ],
    out_specs=[
        pl.BlockSpec((RB, L), lambda i: (i, 0)),
        pl.BlockSpec((RB, OUT_CH), lambda i: (i, 0)),
        pl.BlockSpec((RB, OUT_CH), lambda i: (i, 0)),
    ],
    out_shape=[
        jax.ShapeDtypeStruct((NP, L), jnp.float32),
        jax.ShapeDtypeStruct((NP, OUT_CH), jnp.float32),
        jax.ShapeDtypeStruct((NP, OUT_CH), jnp.float32),
    ],
)


def _mm2_body(acca_ref, accb_ref, u1a_ref, u1b_ref, dinv_ref, b1_ref,
              w2_ref, u2_ref):
    dinv = dinv_ref[...][:, :1]
    b1 = b1_ref[...]
    w2 = w2_ref[...]
    za = jnp.maximum(
        dinv * (acca_ref[0] + acca_ref[1] + u1a_ref[...]) + b1[:, :OUT_CH],
        0.0)
    zb = jnp.maximum(
        dinv * (accb_ref[0] + accb_ref[1] + u1b_ref[...]) + b1[:, OUT_CH:],
        0.0)
    zw = (jnp.dot(za, w2[:OUT_CH, :],
                  preferred_element_type=jnp.float32)
          + jnp.dot(zb, w2[OUT_CH:, :],
                    preferred_element_type=jnp.float32))
    u2_ref[...] = dinv * zw


_mm2 = pl.pallas_call(
    _mm2_body,
    grid=_GRID,
    in_specs=[
        pl.BlockSpec((NC, RB, OUT_CH), lambda i: (0, i, 0)),
        pl.BlockSpec((NC, RB, OUT_CH), lambda i: (0, i, 0)),
        pl.BlockSpec((RB, OUT_CH), lambda i: (i, 0)),
        pl.BlockSpec((RB, OUT_CH), lambda i: (i, 0)),
        pl.BlockSpec((RB, L), lambda i: (i, 0)),
        pl.BlockSpec((1, HID_CH), lambda i: (0, 0)),
        pl.BlockSpec((HID_CH, OUT_CH), lambda i: (0, 0)),
    ],
    out_specs=pl.BlockSpec((RB, OUT_CH), lambda i: (i, 0)),
    out_shape=jax.ShapeDtypeStruct((NP, OUT_CH), jnp.float32),
)


def _fin_body(acc_ref, u2_ref, dinv_ref, b2_ref, z2_ref):
    a = acc_ref[0] + acc_ref[1] + u2_ref[...]
    z2_ref[...] = dinv_ref[...][:, :1] * a + b2_ref[...]


_fin = pl.pallas_call(
    _fin_body,
    grid=_GRID,
    in_specs=[
        pl.BlockSpec((NC, RB, OUT_CH), lambda i: (0, i, 0)),
        pl.BlockSpec((RB, OUT_CH), lambda i: (i, 0)),
        pl.BlockSpec((RB, L), lambda i: (i, 0)),
        pl.BlockSpec((1, OUT_CH), lambda i: (0, 0)),
    ],
    out_specs=pl.BlockSpec((RB, OUT_CH), lambda i: (i, 0)),
    out_shape=jax.ShapeDtypeStruct((NP, OUT_CH), jnp.float32),
)


@jax.jit
def kernel(x, edge_index, edge_label_index, W1, b1, W2, b2):
    ar = jnp.arange(EP - E, dtype=jnp.int32)
    pad_src = (ar * 97) % N
    pad_dst = N + (ar % (NP - N))

    def tile_idx(a, pad):
        return jnp.concatenate([a, pad]).reshape(NW, EBT, BLK)

    src_t = tile_idx(edge_index[0], pad_src)
    dst_t = tile_idx(edge_index[1], pad_dst)
    s_t = tile_idx(edge_label_index[0], pad_src)
    d_t = tile_idx(edge_label_index[1], (ar * 131) % N)

    x_pad = jnp.zeros((NP, IN_CH), jnp.float32).at[:N].set(x)
    z64 = jnp.zeros((NP, OUT_CH), jnp.float32)

    deg_parts = _deg_kernel(dst_t)
    dinv, u1a, u1b = _mm1(deg_parts, x_pad, W1)
    acc1a = _prop64(u1a, src_t, dst_t, z64)
    acc1b = _prop64(u1b, src_t, dst_t, z64)
    u2 = _mm2(acc1a, acc1b, u1a, u1b, dinv, b1.reshape(1, HID_CH), W2)
    acc2 = _prop64(u2, src_t, dst_t, z64)
    z2 = _fin(acc2, u2, dinv, b2.reshape(1, OUT_CH))
    out = _decode_kernel(z2, s_t, d_t)
    return out.reshape(-1)[:E]


# speedup vs baseline: 1.0854x; 1.0854x over previous
import functools

import jax
import jax.numpy as jnp
from jax import lax
from jax.experimental import pallas as pl
from jax.experimental.pallas import tpu as pltpu
from jax.experimental.pallas import tpu_sc as plsc

N = 10000
NP = 10240
E = 320000
IN_CH = 128
HID_CH = 128
OUT_CH = 64

NC = 2
NS = 16
NW = NC * NS
L = 16

BLK = 128
EP = 327680
EBT = EP // (NW * BLK)
ROWS_PT = NP // NS

_mesh = plsc.VectorSubcoreMesh(core_axis_name="c", subcore_axis_name="s")
_SC_PARAMS = pltpu.CompilerParams(use_tc_tiling_on_sc=False,
                                  needs_layout_passes=False)


@functools.partial(
    pl.kernel,
    out_type=jax.ShapeDtypeStruct((NC, NP, L), jnp.float32),
    mesh=_mesh,
    compiler_params=_SC_PARAMS,
    scratch_types=[
        pltpu.VMEM((EBT, BLK), jnp.int32),
        pltpu.VMEM((BLK, L), jnp.float32),
        pltpu.VMEM_SHARED((NP, L), jnp.float32),
    ],
)
def _deg_kernel(dst_hbm, zeros_hbm, deg_out, idx_v, ones_v, acc_sh):
    c = lax.axis_index("c")
    s = lax.axis_index("s")
    wid = s * NC + c

    @pl.loop(0, BLK)
    def _(i):
        ones_v[i, :] = jnp.full((L,), 1.0, jnp.float32)

    row0 = s * ROWS_PT
    pltpu.sync_copy(zeros_hbm.at[pl.ds(row0, ROWS_PT)],
                    acc_sh.at[pl.ds(row0, ROWS_PT)])
    pltpu.sync_copy(dst_hbm.at[wid], idx_v)
    plsc.subcore_barrier()

    @pl.loop(0, EBT)
    def _(j):
        pltpu.sync_copy(ones_v, acc_sh.at[idx_v.at[j]], add=True)

    plsc.subcore_barrier()
    pltpu.sync_copy(acc_sh.at[pl.ds(row0, ROWS_PT)],
                    deg_out.at[c].at[pl.ds(row0, ROWS_PT)])


def _make_prop(D, NBUF):
    @functools.partial(
        pl.kernel,
        out_type=jax.ShapeDtypeStruct((NC, NP, D), jnp.float32),
        mesh=_mesh,
        compiler_params=_SC_PARAMS,
        scratch_types=(
            [pltpu.VMEM((EBT, BLK), jnp.int32),
             pltpu.VMEM((EBT, BLK), jnp.int32)]
            + [pltpu.VMEM((BLK, D), jnp.float32)] * NBUF
            + [pltpu.VMEM_SHARED((NP, D), jnp.float32)]
            + [pltpu.SemaphoreType.DMA] * (2 * NBUF)
        ),
    )
    def _prop(u_hbm, src_hbm, dst_hbm, zeros_hbm, acc_out, si_v, di_v, *bufs):
        rows = bufs[:NBUF]
        acc_sh = bufs[NBUF]
        sem_g = bufs[NBUF + 1:2 * NBUF + 1]
        sem_s = bufs[2 * NBUF + 1:3 * NBUF + 1]
        c = lax.axis_index("c")
        s = lax.axis_index("s")
        wid = s * NC + c
        row0 = s * ROWS_PT
        pltpu.sync_copy(zeros_hbm.at[pl.ds(row0, ROWS_PT)],
                        acc_sh.at[pl.ds(row0, ROWS_PT)])
        pltpu.sync_copy(src_hbm.at[wid], si_v)
        pltpu.sync_copy(dst_hbm.at[wid], di_v)
        plsc.subcore_barrier()

        for b in range(NBUF):
            pltpu.async_copy(u_hbm.at[si_v.at[b]], rows[b], sem_g[b])

        @pl.loop(0, EBT - NBUF, step=NBUF)
        def _(j0):
            scatters = []
            for b in range(NBUF):
                jb = j0 + b
                pltpu.make_async_copy(u_hbm.at[si_v.at[jb]],
                                      rows[b], sem_g[b]).wait()
                scatters.append(
                    pltpu.async_copy(rows[b], acc_sh.at[di_v.at[jb]],
                                     sem_s[b], add=True))
            for b in range(NBUF):
                scatters[b].wait()
                pltpu.async_copy(u_hbm.at[si_v.at[j0 + b + NBUF]],
                                 rows[b], sem_g[b])

        for b in range(NBUF):
            jb = EBT - NBUF + b
            pltpu.make_async_copy(u_hbm.at[si_v.at[jb]],
                                  rows[b], sem_g[b]).wait()
            pltpu.async_copy(rows[b], acc_sh.at[di_v.at[jb]],
                             sem_s[b], add=True).wait()

        plsc.subcore_barrier()
        pltpu.sync_copy(acc_sh.at[pl.ds(row0, ROWS_PT)],
                        acc_out.at[c].at[pl.ds(row0, ROWS_PT)])

    return _prop


_prop64 = _make_prop(OUT_CH, 8)


@functools.partial(
    pl.kernel,
    out_type=jax.ShapeDtypeStruct((NW, EBT, BLK), jnp.float32),
    mesh=_mesh,
    compiler_params=_SC_PARAMS,
    scratch_types=(
        [pltpu.VMEM((EBT, BLK), jnp.int32),
         pltpu.VMEM((EBT, BLK), jnp.int32),
         pltpu.VMEM((EBT, BLK), jnp.float32)]
        + [pltpu.VMEM((BLK, OUT_CH), jnp.float32)] * 8
        + [pltpu.SemaphoreType.DMA] * 8
    ),
)
def _decode_kernel(z2_hbm, s_hbm, d_hbm, out_hbm, si_v, di_v, o_v, *bufs):
    DB = 4
    zs = bufs[:DB]
    zd = bufs[DB:2 * DB]
    sem_s = bufs[2 * DB:3 * DB]
    sem_d = bufs[3 * DB:]
    c = lax.axis_index("c")
    s = lax.axis_index("s")
    wid = s * NC + c
    pltpu.sync_copy(s_hbm.at[wid], si_v)
    pltpu.sync_copy(d_hbm.at[wid], di_v)
    row0 = s * ROWS_PT

    def scoped(z2_sh):
        last_lane = lax.iota(jnp.int32, L) == (L - 1)

        def compute_block(jb, b):
            jb_vec = jnp.full((L,), jb, jnp.int32)

            @pl.loop(0, BLK, step=8)
            def _(e0):
                e0_vec = jnp.full((L,), e0, jnp.int32)
                for k in range(8):
                    e = e0 + k
                    p = zs[b][e, pl.ds(0, L)] * zd[b][e, pl.ds(0, L)]
                    for q in range(1, OUT_CH // L):
                        p = p + (zs[b][e, pl.ds(q * L, L)]
                                 * zd[b][e, pl.ds(q * L, L)])
                    tot = plsc.cumsum(p)
                    plsc.store_scatter(o_v, [jb_vec, e0_vec + k],
                                       tot, mask=last_lane)

        def wait_block(jb, b):
            pltpu.make_async_copy(z2_sh.at[si_v.at[jb]], zs[b],
                                  sem_s[b]).wait()
            pltpu.make_async_copy(z2_sh.at[di_v.at[jb]], zd[b],
                                  sem_d[b]).wait()

        for b in range(DB):
            pltpu.async_copy(z2_sh.at[si_v.at[b]], zs[b], sem_s[b])
            pltpu.async_copy(z2_sh.at[di_v.at[b]], zd[b], sem_d[b])

        @pl.loop(0, EBT - DB, step=DB)
        def _(j0):
            for b in range(DB):
                jb = j0 + b
                wait_block(jb, b)
                compute_block(jb, b)
                jn = jb + DB
                pltpu.async_copy(z2_sh.at[si_v.at[jn]], zs[b], sem_s[b])
                pltpu.async_copy(z2_sh.at[di_v.at[jn]], zd[b], sem_d[b])

        for b in range(DB):
            jb = EBT - DB + b
            wait_block(jb, b)
            compute_block(jb, b)

        pltpu.sync_copy(o_v, out_hbm.at[wid])

    scoped(z2_hbm)


RB = 512
_GRID = (NP // RB,)
_HI = jax.lax.Precision.HIGHEST


def _mm1_body(deg_ref, x_ref, w1_ref, dinv_ref, u1a_ref, u1b_ref):
    deg = deg_ref[0] + deg_ref[1] + 1.0
    dinv = jax.lax.rsqrt(deg)
    dinv_ref[...] = dinv
    xw = jnp.dot(x_ref[...], w1_ref[...],
                 preferred_element_type=jnp.float32)
    u1 = dinv[:, :1] * xw
    u1a_ref[...] = u1[:, :OUT_CH]
    u1b_ref[...] = u1[:, OUT_CH:]


_mm1 = pl.pallas_call(
    _mm1_body,
    grid=_GRID,
    in_specs=[
        pl.BlockSpec((NC, RB, L), lambda i: (0, i, 0)),
        pl.BlockSpec((RB, IN_CH), lambda i: (i, 0)),
        pl.BlockSpec((IN_CH, HID_CH), lambda i: (0, 0)),
    ],
    out_specs=[
        pl.BlockSpec((RB, L), lambda i: (i, 0)),
        pl.BlockSpec((RB, OUT_CH), lambda i: (i, 0)),
        pl.BlockSpec((RB, OUT_CH), lambda i: (i, 0)),
    ],
    out_shape=[
        jax.ShapeDtypeStruct((NP, L), jnp.float32),
        jax.ShapeDtypeStruct((NP, OUT_CH), jnp.float32),
        jax.ShapeDtypeStruct((NP, OUT_CH), jnp.float32),
    ],
)


def _mm2_body(acca_ref, accb_ref, u1a_ref, u1b_ref, dinv_ref, b1_ref,
              w2_ref, u2_ref):
    dinv = dinv_ref[...][:, :1]
    b1 = b1_ref[...]
    w2 = w2_ref[...]
    za = jnp.maximum(
        dinv * (acca_ref[0] + acca_ref[1] + u1a_ref[...]) + b1[:, :OUT_CH],
        0.0)
    zb = jnp.maximum(
        dinv * (accb_ref[0] + accb_ref[1] + u1b_ref[...]) + b1[:, OUT_CH:],
        0.0)
    zw = (jnp.dot(za, w2[:OUT_CH, :],
                  preferred_element_type=jnp.float32)
          + jnp.dot(zb, w2[OUT_CH:, :],
                    preferred_element_type=jnp.float32))
    u2_ref[...] = dinv * zw


_mm2 = pl.pallas_call(
    _mm2_body,
    grid=_GRID,
    in_specs=[
        pl.BlockSpec((NC, RB, OUT_CH), lambda i: (0, i, 0)),
        pl.BlockSpec((NC, RB, OUT_CH), lambda i: (0, i, 0)),
        pl.BlockSpec((RB, OUT_CH), lambda i: (i, 0)),
        pl.BlockSpec((RB, OUT_CH), lambda i: (i, 0)),
        pl.BlockSpec((RB, L), lambda i: (i, 0)),
        pl.BlockSpec((1, HID_CH), lambda i: (0, 0)),
        pl.BlockSpec((HID_CH, OUT_CH), lambda i: (0, 0)),
    ],
    out_specs=pl.BlockSpec((RB, OUT_CH), lambda i: (i, 0)),
    out_shape=jax.ShapeDtypeStruct((NP, OUT_CH), jnp.float32),
)


def _fin_body(acc_ref, u2_ref, dinv_ref, b2_ref, z2_ref):
    a = acc_ref[0] + acc_ref[1] + u2_ref[...]
    z2_ref[...] = dinv_ref[...][:, :1] * a + b2_ref[...]


_fin = pl.pallas_call(
    _fin_body,
    grid=_GRID,
    in_specs=[
        pl.BlockSpec((NC, RB, OUT_CH), lambda i: (0, i, 0)),
        pl.BlockSpec((RB, OUT_CH), lambda i: (i, 0)),
        pl.BlockSpec((RB, L), lambda i: (i, 0)),
        pl.BlockSpec((1, OUT_CH), lambda i: (0, 0)),
    ],
    out_specs=pl.BlockSpec((RB, OUT_CH), lambda i: (i, 0)),
    out_shape=jax.ShapeDtypeStruct((NP, OUT_CH), jnp.float32),
)


@jax.jit
def kernel(x, edge_index, edge_label_index, W1, b1, W2, b2):
    ar = jnp.arange(EP - E, dtype=jnp.int32)
    pad_src = (ar * 97) % N
    pad_dst = N + (ar % (NP - N))

    def tile_idx(a, pad):
        return jnp.concatenate([a, pad]).reshape(NW, EBT, BLK)

    src_t = tile_idx(edge_index[0], pad_src)
    dst_t = tile_idx(edge_index[1], pad_dst)
    s_t = tile_idx(edge_label_index[0], pad_src)
    d_t = tile_idx(edge_label_index[1], (ar * 131) % N)

    x_pad = jnp.zeros((NP, IN_CH), jnp.float32).at[:N].set(x)
    z16 = jnp.zeros((NP, L), jnp.float32)
    z64 = jnp.zeros((NP, OUT_CH), jnp.float32)

    deg_parts = _deg_kernel(dst_t, z16)
    dinv, u1a, u1b = _mm1(deg_parts, x_pad, W1)
    acc1a = _prop64(u1a, src_t, dst_t, z64)
    acc1b = _prop64(u1b, src_t, dst_t, z64)
    u2 = _mm2(acc1a, acc1b, u1a, u1b, dinv, b1.reshape(1, HID_CH), W2)
    acc2 = _prop64(u2, src_t, dst_t, z64)
    z2 = _fin(acc2, u2, dinv, b2.reshape(1, OUT_CH))
    out = _decode_kernel(z2, s_t, d_t)
    return out.reshape(-1)[:E]
